# Initial kernel scaffold; baseline (speedup 1.0000x reference)
import jax
import jax.numpy as jnp
from jax.experimental import pallas as pl


def kernel(tgt, query_pos, reference_points, src, src_spatial_shapes, level_start_index, inner_w):
    raise NotImplementedError("write your pallas kernel here")


# speedup vs baseline: 1.0841x; 1.0841x over previous
import math

import numpy as np

import jax
import jax.numpy as jnp
from jax.experimental import pallas as pl
from jax.experimental.pallas import tpu as pltpu

_D = 128
_C = 8
_MAXC = 16
_TB = 256


def _insert_rows():
    d_model = 2 * _D
    position = np.arange(0, _MAXC, dtype=np.float32)[:, None]
    div_term = np.exp(
        np.arange(0, d_model, 2, dtype=np.float32) * (-math.log(10000.0) / d_model)
    )
    pe = np.zeros((_MAXC, d_model), dtype=np.float32)
    pe[:, 0::2] = np.sin(position * div_term)
    pe[:, 1::2] = np.cos(position * div_term)
    return pe[1::2]


def _body(tgt_ref, qp_ref, ins_ref, w_ref, inner_ref, ntgt_ref, nqp_ref):
    tb = tgt_ref.shape[0]
    tgt = tgt_ref[...]
    qp = qp_ref[...]
    w = w_ref[...]
    ins = ins_ref[...]
    ins_qp = ins[:, :_D]
    ins_tgt = ins[:, _D:]

    ins_inner = ins_tgt + jnp.dot(
        ins_tgt + ins_qp, w, preferred_element_type=jnp.float32
    )

    x = tgt + qp
    for c in range(_C):
        src = slice(c * _D, (c + 1) * _D)
        dst_o = slice(2 * c * _D, (2 * c + 1) * _D)
        dst_i = slice((2 * c + 1) * _D, (2 * c + 2) * _D)

        y = jnp.dot(x[:, src], w, preferred_element_type=jnp.float32)
        inner_ref[:, dst_o] = tgt[:, src] + y
        inner_ref[:, dst_i] = jnp.broadcast_to(ins_inner[c : c + 1, :], (tb, _D))
        ntgt_ref[:, dst_o] = tgt[:, src]
        ntgt_ref[:, dst_i] = jnp.broadcast_to(ins_tgt[c : c + 1, :], (tb, _D))
        nqp_ref[:, dst_o] = qp[:, src]
        nqp_ref[:, dst_i] = jnp.broadcast_to(ins_qp[c : c + 1, :], (tb, _D))


def kernel(tgt, query_pos, reference_points, src, src_spatial_shapes,
           level_start_index, inner_w):
    B, Q, C, D = tgt.shape
    N = B * Q
    CD = C * D
    CD2 = 2 * CD

    tgt2 = tgt.reshape(N, CD)
    qp2 = query_pos.reshape(N, CD)
    ins = jnp.asarray(_insert_rows())

    tb = min(_TB, N)
    n_pad = ((N + tb - 1) // tb) * tb
    if n_pad != N:
        pad = ((0, n_pad - N), (0, 0))
        tgt2 = jnp.pad(tgt2, pad)
        qp2 = jnp.pad(qp2, pad)

    inner2, ntgt2, nqp2 = pl.pallas_call(
        _body,
        grid=(n_pad // tb,),
        in_specs=[
            pl.BlockSpec((tb, CD), lambda i: (i, 0)),
            pl.BlockSpec((tb, CD), lambda i: (i, 0)),
            pl.BlockSpec((C, 2 * D), lambda i: (0, 0)),
            pl.BlockSpec((D, D), lambda i: (0, 0)),
        ],
        out_specs=(
            pl.BlockSpec((tb, CD2), lambda i: (i, 0)),
            pl.BlockSpec((tb, CD2), lambda i: (i, 0)),
            pl.BlockSpec((tb, CD2), lambda i: (i, 0)),
        ),
        out_shape=(
            jax.ShapeDtypeStruct((n_pad, CD2), jnp.float32),
            jax.ShapeDtypeStruct((n_pad, CD2), jnp.float32),
            jax.ShapeDtypeStruct((n_pad, CD2), jnp.float32),
        ),
        compiler_params=pltpu.CompilerParams(dimension_semantics=("parallel",)),
    )(tgt2, qp2, ins, inner_w)

    inner_out = inner2[:N].reshape(B, Q, 2 * C, D)
    new_tgt = ntgt2[:N].reshape(B, Q, 2 * C, D)
    new_qp = nqp2[:N].reshape(B, Q, 2 * C, D)

    L = reference_points.shape[-2]
    rp_f = reference_points.reshape(N, C, L, 2)
    ins_rp = ((rp_f + jnp.roll(rp_f, -1, axis=1)) * 0.5).reshape(B, Q, C, L, 2)

    return inner_out, new_tgt, new_qp, ins_rp


# speedup vs baseline: 2.6461x; 2.4408x over previous
import math

import numpy as np

import jax
import jax.numpy as jnp
from jax.experimental import pallas as pl
from jax.experimental.pallas import tpu as pltpu

_D = 128
_C = 8
_MAXC = 16
_TB = 256


def _insert_rows():
    d_model = 2 * _D
    position = np.arange(0, _MAXC, dtype=np.float32)[:, None]
    div_term = np.exp(
        np.arange(0, d_model, 2, dtype=np.float32) * (-math.log(10000.0) / d_model)
    )
    pe = np.zeros((_MAXC, d_model), dtype=np.float32)
    pe[:, 0::2] = np.sin(position * div_term)
    pe[:, 1::2] = np.cos(position * div_term)
    return pe[1::2]


def _body(tgt_ref, qp_ref, ins_ref, w_ref, inner_ref, ntgt_ref, nqp_ref):
    tb = tgt_ref.shape[0]
    tgt = tgt_ref[...]
    qp = qp_ref[...]
    w = w_ref[...]
    ins = ins_ref[...]
    ins_qp = ins[:, :_D]
    ins_tgt = ins[:, _D:]

    ins_inner = ins_tgt + jnp.dot(
        ins_tgt + ins_qp, w, preferred_element_type=jnp.float32
    )

    x2 = (tgt + qp).reshape(tb * _C, _D)
    y = jnp.dot(x2, w, preferred_element_type=jnp.float32)
    inner_orig = tgt + y.reshape(tb, _C, _D)

    for c in range(_C):
        inner_ref[:, 2 * c, :] = inner_orig[:, c, :]
        inner_ref[:, 2 * c + 1, :] = jnp.broadcast_to(ins_inner[c, :], (tb, _D))
        ntgt_ref[:, 2 * c, :] = tgt[:, c, :]
        ntgt_ref[:, 2 * c + 1, :] = jnp.broadcast_to(ins_tgt[c, :], (tb, _D))
        nqp_ref[:, 2 * c, :] = qp[:, c, :]
        nqp_ref[:, 2 * c + 1, :] = jnp.broadcast_to(ins_qp[c, :], (tb, _D))


def kernel(tgt, query_pos, reference_points, src, src_spatial_shapes,
           level_start_index, inner_w):
    B, Q, C, D = tgt.shape
    N = B * Q

    tgt3 = tgt.reshape(N, C, D)
    qp3 = query_pos.reshape(N, C, D)
    ins = jnp.asarray(_insert_rows())

    tb = min(_TB, N)
    n_pad = ((N + tb - 1) // tb) * tb
    if n_pad != N:
        pad = ((0, n_pad - N), (0, 0), (0, 0))
        tgt3 = jnp.pad(tgt3, pad)
        qp3 = jnp.pad(qp3, pad)

    inner3, ntgt3, nqp3 = pl.pallas_call(
        _body,
        grid=(n_pad // tb,),
        in_specs=[
            pl.BlockSpec((tb, C, D), lambda i: (i, 0, 0)),
            pl.BlockSpec((tb, C, D), lambda i: (i, 0, 0)),
            pl.BlockSpec((C, 2 * D), lambda i: (0, 0)),
            pl.BlockSpec((D, D), lambda i: (0, 0)),
        ],
        out_specs=(
            pl.BlockSpec((tb, 2 * C, D), lambda i: (i, 0, 0)),
            pl.BlockSpec((tb, 2 * C, D), lambda i: (i, 0, 0)),
            pl.BlockSpec((tb, 2 * C, D), lambda i: (i, 0, 0)),
        ),
        out_shape=(
            jax.ShapeDtypeStruct((n_pad, 2 * C, D), jnp.float32),
            jax.ShapeDtypeStruct((n_pad, 2 * C, D), jnp.float32),
            jax.ShapeDtypeStruct((n_pad, 2 * C, D), jnp.float32),
        ),
        compiler_params=pltpu.CompilerParams(dimension_semantics=("parallel",)),
    )(tgt3, qp3, ins, inner_w)

    inner_out = inner3[:N].reshape(B, Q, 2 * C, D)
    new_tgt = ntgt3[:N].reshape(B, Q, 2 * C, D)
    new_qp = nqp3[:N].reshape(B, Q, 2 * C, D)

    L = reference_points.shape[-2]
    rp_f = reference_points.reshape(N, C, L, 2)
    ins_rp = ((rp_f + jnp.roll(rp_f, -1, axis=1)) * 0.5).reshape(B, Q, C, L, 2)

    return inner_out, new_tgt, new_qp, ins_rp


# speedup vs baseline: 3.4804x; 1.3153x over previous
import math

import numpy as np

import jax
import jax.numpy as jnp
from jax.experimental import pallas as pl
from jax.experimental.pallas import tpu as pltpu

_D = 128
_C = 8
_MAXC = 16
_TB = 256


def _insert_rows():
    d_model = 2 * _D
    position = np.arange(0, _MAXC, dtype=np.float32)[:, None]
    div_term = np.exp(
        np.arange(0, d_model, 2, dtype=np.float32) * (-math.log(10000.0) / d_model)
    )
    pe = np.zeros((_MAXC, d_model), dtype=np.float32)
    pe[:, 0::2] = np.sin(position * div_term)
    pe[:, 1::2] = np.cos(position * div_term)
    return pe[1::2]


def _body(tgt_ref, qp_ref, ins_tgt_ref, ins_qp_ref, w_ref,
          inner_ref, ntgt_ref, nqp_ref):
    tb = tgt_ref.shape[0]
    tgt = tgt_ref[...]
    qp = qp_ref[...]
    w = w_ref[...]
    ins_tgt = ins_tgt_ref[...]
    ins_qp = ins_qp_ref[...]

    ins_inner = ins_tgt + jnp.dot(
        ins_tgt + ins_qp, w, preferred_element_type=jnp.float32
    )

    x2 = (tgt + qp).reshape(tb * _C, _D)
    y = jnp.dot(x2, w, preferred_element_type=jnp.float32)
    inner_orig = tgt + y.reshape(tb, _C, _D)

    ev = pl.ds(0, _C, 2)
    od = pl.ds(1, _C, 2)
    inner_ref[:, ev, :] = inner_orig
    inner_ref[:, od, :] = jnp.broadcast_to(ins_inner[None], (tb, _C, _D))
    ntgt_ref[:, ev, :] = tgt
    ntgt_ref[:, od, :] = jnp.broadcast_to(ins_tgt[None], (tb, _C, _D))
    nqp_ref[:, ev, :] = qp
    nqp_ref[:, od, :] = jnp.broadcast_to(ins_qp[None], (tb, _C, _D))


def kernel(tgt, query_pos, reference_points, src, src_spatial_shapes,
           level_start_index, inner_w):
    B, Q, C, D = tgt.shape
    N = B * Q

    tgt3 = tgt.reshape(N, C, D)
    qp3 = query_pos.reshape(N, C, D)
    ins_rows = _insert_rows()
    ins_qp_c = jnp.asarray(ins_rows[:, :D])
    ins_tgt_c = jnp.asarray(np.ascontiguousarray(ins_rows[:, D:]))

    tb = min(_TB, N)
    n_pad = ((N + tb - 1) // tb) * tb
    if n_pad != N:
        pad = ((0, n_pad - N), (0, 0), (0, 0))
        tgt3 = jnp.pad(tgt3, pad)
        qp3 = jnp.pad(qp3, pad)

    inner3, ntgt3, nqp3 = pl.pallas_call(
        _body,
        grid=(n_pad // tb,),
        in_specs=[
            pl.BlockSpec((tb, C, D), lambda i: (i, 0, 0)),
            pl.BlockSpec((tb, C, D), lambda i: (i, 0, 0)),
            pl.BlockSpec((C, D), lambda i: (0, 0)),
            pl.BlockSpec((C, D), lambda i: (0, 0)),
            pl.BlockSpec((D, D), lambda i: (0, 0)),
        ],
        out_specs=(
            pl.BlockSpec((tb, 2 * C, D), lambda i: (i, 0, 0)),
            pl.BlockSpec((tb, 2 * C, D), lambda i: (i, 0, 0)),
            pl.BlockSpec((tb, 2 * C, D), lambda i: (i, 0, 0)),
        ),
        out_shape=(
            jax.ShapeDtypeStruct((n_pad, 2 * C, D), jnp.float32),
            jax.ShapeDtypeStruct((n_pad, 2 * C, D), jnp.float32),
            jax.ShapeDtypeStruct((n_pad, 2 * C, D), jnp.float32),
        ),
        compiler_params=pltpu.CompilerParams(dimension_semantics=("parallel",)),
    )(tgt3, qp3, ins_tgt_c, ins_qp_c, inner_w)

    inner_out = inner3[:N].reshape(B, Q, 2 * C, D)
    new_tgt = ntgt3[:N].reshape(B, Q, 2 * C, D)
    new_qp = nqp3[:N].reshape(B, Q, 2 * C, D)

    L = reference_points.shape[-2]
    rp_f = reference_points.reshape(N, C, L, 2)
    ins_rp = ((rp_f + jnp.roll(rp_f, -1, axis=1)) * 0.5).reshape(B, Q, C, L, 2)

    return inner_out, new_tgt, new_qp, ins_rp


# speedup vs baseline: 3.5718x; 1.0263x over previous
import math

import numpy as np

import jax
import jax.numpy as jnp
from jax.experimental import pallas as pl
from jax.experimental.pallas import tpu as pltpu

_D = 128
_C = 8
_MAXC = 16
_TB = 512


def _insert_rows():
    d_model = 2 * _D
    position = np.arange(0, _MAXC, dtype=np.float32)[:, None]
    div_term = np.exp(
        np.arange(0, d_model, 2, dtype=np.float32) * (-math.log(10000.0) / d_model)
    )
    pe = np.zeros((_MAXC, d_model), dtype=np.float32)
    pe[:, 0::2] = np.sin(position * div_term)
    pe[:, 1::2] = np.cos(position * div_term)
    return pe[1::2]


def _body(tgt_ref, qp_ref, ins_tgt_ref, ins_qp_ref, w_ref,
          inner_ref, ntgt_ref, nqp_ref):
    tb = tgt_ref.shape[0]
    tgt = tgt_ref[...]
    qp = qp_ref[...]
    w = w_ref[...]
    ins_tgt = ins_tgt_ref[...]
    ins_qp = ins_qp_ref[...]

    ins_inner = ins_tgt + jnp.dot(
        ins_tgt + ins_qp, w, preferred_element_type=jnp.float32
    )

    x2 = (tgt + qp).reshape(tb * _C, _D)
    y = jnp.dot(x2, w, preferred_element_type=jnp.float32)
    inner_orig = tgt + y.reshape(tb, _C, _D)

    ev = pl.ds(0, _C, 2)
    od = pl.ds(1, _C, 2)
    inner_ref[:, ev, :] = inner_orig
    inner_ref[:, od, :] = jnp.broadcast_to(ins_inner[None], (tb, _C, _D))
    ntgt_ref[:, ev, :] = tgt
    ntgt_ref[:, od, :] = jnp.broadcast_to(ins_tgt[None], (tb, _C, _D))
    nqp_ref[:, ev, :] = qp
    nqp_ref[:, od, :] = jnp.broadcast_to(ins_qp[None], (tb, _C, _D))


def kernel(tgt, query_pos, reference_points, src, src_spatial_shapes,
           level_start_index, inner_w):
    B, Q, C, D = tgt.shape
    N = B * Q

    tgt3 = tgt.reshape(N, C, D)
    qp3 = query_pos.reshape(N, C, D)
    ins_rows = _insert_rows()
    ins_qp_c = jnp.asarray(ins_rows[:, :D])
    ins_tgt_c = jnp.asarray(np.ascontiguousarray(ins_rows[:, D:]))

    tb = min(_TB, N)
    n_pad = ((N + tb - 1) // tb) * tb
    if n_pad != N:
        pad = ((0, n_pad - N), (0, 0), (0, 0))
        tgt3 = jnp.pad(tgt3, pad)
        qp3 = jnp.pad(qp3, pad)

    inner3, ntgt3, nqp3 = pl.pallas_call(
        _body,
        grid=(n_pad // tb,),
        in_specs=[
            pl.BlockSpec((tb, C, D), lambda i: (i, 0, 0)),
            pl.BlockSpec((tb, C, D), lambda i: (i, 0, 0)),
            pl.BlockSpec((C, D), lambda i: (0, 0)),
            pl.BlockSpec((C, D), lambda i: (0, 0)),
            pl.BlockSpec((D, D), lambda i: (0, 0)),
        ],
        out_specs=(
            pl.BlockSpec((tb, 2 * C, D), lambda i: (i, 0, 0)),
            pl.BlockSpec((tb, 2 * C, D), lambda i: (i, 0, 0)),
            pl.BlockSpec((tb, 2 * C, D), lambda i: (i, 0, 0)),
        ),
        out_shape=(
            jax.ShapeDtypeStruct((n_pad, 2 * C, D), jnp.float32),
            jax.ShapeDtypeStruct((n_pad, 2 * C, D), jnp.float32),
            jax.ShapeDtypeStruct((n_pad, 2 * C, D), jnp.float32),
        ),
        compiler_params=pltpu.CompilerParams(dimension_semantics=("parallel",)),
    )(tgt3, qp3, ins_tgt_c, ins_qp_c, inner_w)

    inner_out = inner3[:N].reshape(B, Q, 2 * C, D)
    new_tgt = ntgt3[:N].reshape(B, Q, 2 * C, D)
    new_qp = nqp3[:N].reshape(B, Q, 2 * C, D)

    L = reference_points.shape[-2]
    rp_f = reference_points.reshape(N, C, L, 2)
    ins_rp = ((rp_f + jnp.roll(rp_f, -1, axis=1)) * 0.5).reshape(B, Q, C, L, 2)

    return inner_out, new_tgt, new_qp, ins_rp
